# Initial kernel scaffold; baseline (speedup 1.0000x reference)
import jax
import jax.numpy as jnp
from jax.experimental import pallas as pl


def kernel(rgap, sgap, pcount, W):
    raise NotImplementedError("write your pallas kernel here")


# speedup vs baseline: 4.8780x; 4.8780x over previous
import jax
import jax.numpy as jnp
from jax import lax
from jax.experimental import pallas as pl

B, T = 1024, 200
N = B * T
NRG, NSG, NPC, EMB = 32, 32, 64, 128
BT = 2048
GRID = N // BT


def _tc_body(r_ref, s_ref, p_ref, wt_ref, r_oh_ref, s_oh_ref, p_oh_ref, emb_ref):
    r = r_ref[...]
    s = s_ref[...]
    p = p_ref[...]

    def oh(idx, n):
        return (lax.broadcasted_iota(jnp.int32, (BT, n), 1) == idx).astype(jnp.float32)

    r_oh_ref[...] = oh(r, NRG)
    s_oh_ref[...] = oh(s, NSG)
    p_oh_ref[...] = oh(p, NPC)
    i128 = lax.broadcasted_iota(jnp.int32, (BT, EMB), 1)
    tg = ((i128 == r) | (i128 == s + NRG) | (i128 == p + NRG + NSG)).astype(jnp.float32)
    emb_ref[...] = jnp.dot(tg, wt_ref[...], preferred_element_type=jnp.float32)


def kernel(rgap, sgap, pcount, W):
    r = rgap.reshape(N, 1)
    s = sgap.reshape(N, 1)
    p = pcount.reshape(N, 1)
    wt = W.T

    idx_spec = pl.BlockSpec((BT, 1), lambda i: (i, 0))
    r_oh, s_oh, p_oh, emb = pl.pallas_call(
        _tc_body,
        grid=(GRID,),
        in_specs=[idx_spec, idx_spec, idx_spec,
                  pl.BlockSpec((EMB, EMB), lambda i: (0, 0))],
        out_specs=[pl.BlockSpec((BT, NRG), lambda i: (i, 0)),
                   pl.BlockSpec((BT, NSG), lambda i: (i, 0)),
                   pl.BlockSpec((BT, NPC), lambda i: (i, 0)),
                   pl.BlockSpec((BT, EMB), lambda i: (i, 0))],
        out_shape=[jax.ShapeDtypeStruct((N, NRG), jnp.float32),
                   jax.ShapeDtypeStruct((N, NSG), jnp.float32),
                   jax.ShapeDtypeStruct((N, NPC), jnp.float32),
                   jax.ShapeDtypeStruct((N, EMB), jnp.float32)],
    )(r, s, p, wt)
    return (r_oh.reshape(B, T, NRG), s_oh.reshape(B, T, NSG),
            p_oh.reshape(B, T, NPC), emb.reshape(B, T, EMB))


# speedup vs baseline: 7.5236x; 1.5424x over previous
import jax
import jax.numpy as jnp
from jax import lax
from jax.experimental import pallas as pl

B, T = 1024, 200
NRG, NSG, NPC, EMB = 32, 32, 64, 128
BB = 16
GRID = B // BB


def _tc_body(r_ref, s_ref, p_ref, wt_ref, r_oh_ref, s_oh_ref, p_oh_ref, emb_ref):
    r = r_ref[...][:, :, None]
    s = s_ref[...][:, :, None]
    p = p_ref[...][:, :, None]

    def oh(idx, n):
        return (lax.broadcasted_iota(jnp.int32, (BB, T, n), 2) == idx
                ).astype(jnp.float32)

    r_oh_ref[...] = oh(r, NRG)
    s_oh_ref[...] = oh(s, NSG)
    p_oh_ref[...] = oh(p, NPC)
    i128 = lax.broadcasted_iota(jnp.int32, (BB, T, EMB), 2)
    tg = ((i128 == r) | (i128 == s + NRG) | (i128 == p + NRG + NSG)
          ).astype(jnp.float32)
    emb = jnp.dot(tg.reshape(BB * T, EMB), wt_ref[...],
                  preferred_element_type=jnp.float32)
    emb_ref[...] = emb.reshape(BB, T, EMB)


def kernel(rgap, sgap, pcount, W):
    wt = W.T

    idx_spec = pl.BlockSpec((BB, T), lambda i: (i, 0))
    r_oh, s_oh, p_oh, emb = pl.pallas_call(
        _tc_body,
        grid=(GRID,),
        in_specs=[idx_spec, idx_spec, idx_spec,
                  pl.BlockSpec((EMB, EMB), lambda i: (0, 0))],
        out_specs=[pl.BlockSpec((BB, T, NRG), lambda i: (i, 0, 0)),
                   pl.BlockSpec((BB, T, NSG), lambda i: (i, 0, 0)),
                   pl.BlockSpec((BB, T, NPC), lambda i: (i, 0, 0)),
                   pl.BlockSpec((BB, T, EMB), lambda i: (i, 0, 0))],
        out_shape=[jax.ShapeDtypeStruct((B, T, NRG), jnp.float32),
                   jax.ShapeDtypeStruct((B, T, NSG), jnp.float32),
                   jax.ShapeDtypeStruct((B, T, NPC), jnp.float32),
                   jax.ShapeDtypeStruct((B, T, EMB), jnp.float32)],
    )(rgap, sgap, pcount, wt)
    return (r_oh, s_oh, p_oh, emb)


# speedup vs baseline: 16.5424x; 2.1987x over previous
import jax
import jax.numpy as jnp
from jax import lax
from jax.experimental import pallas as pl

B, T = 1024, 200
NRG, NSG, NPC, EMB = 32, 32, 64, 128
TB = 8
GRID = T // TB


def _tc_body(r_ref, s_ref, p_ref, w_ref, r_oh_ref, s_oh_ref, p_oh_ref, emb_ref):
    r = r_ref[...][:, None, :]
    s = s_ref[...][:, None, :]
    p = p_ref[...][:, None, :]

    def oh(idx, n):
        return (lax.broadcasted_iota(jnp.int32, (TB, n, B), 1) == idx
                ).astype(jnp.float32)

    r_oh_ref[...] = oh(r, NRG)
    s_oh_ref[...] = oh(s, NSG)
    p_oh_ref[...] = oh(p, NPC)
    i128 = lax.broadcasted_iota(jnp.int32, (TB, EMB, B), 1)
    tg = ((i128 == r) | (i128 == s + NRG) | (i128 == p + NRG + NSG)
          ).astype(jnp.float32)
    w = w_ref[...]
    for tt in range(TB):
        emb_ref[tt] = jnp.dot(w, tg[tt], preferred_element_type=jnp.float32)


def kernel(rgap, sgap, pcount, W):
    rT = rgap.T
    sT = sgap.T
    pT = pcount.T

    idx_spec = pl.BlockSpec((TB, B), lambda i: (i, 0))
    r_oh, s_oh, p_oh, emb = pl.pallas_call(
        _tc_body,
        grid=(GRID,),
        in_specs=[idx_spec, idx_spec, idx_spec,
                  pl.BlockSpec((EMB, EMB), lambda i: (0, 0))],
        out_specs=[pl.BlockSpec((TB, NRG, B), lambda i: (i, 0, 0)),
                   pl.BlockSpec((TB, NSG, B), lambda i: (i, 0, 0)),
                   pl.BlockSpec((TB, NPC, B), lambda i: (i, 0, 0)),
                   pl.BlockSpec((TB, EMB, B), lambda i: (i, 0, 0))],
        out_shape=[jax.ShapeDtypeStruct((T, NRG, B), jnp.float32),
                   jax.ShapeDtypeStruct((T, NSG, B), jnp.float32),
                   jax.ShapeDtypeStruct((T, NPC, B), jnp.float32),
                   jax.ShapeDtypeStruct((T, EMB, B), jnp.float32)],
    )(rT, sT, pT, W)
    return (r_oh.transpose(2, 0, 1), s_oh.transpose(2, 0, 1),
            p_oh.transpose(2, 0, 1), emb.transpose(2, 0, 1))
